# baseline (speedup 1.0000x reference)
import functools

import jax
import jax.numpy as jnp
from jax import lax
from jax.experimental import pallas as pl
from jax.experimental.pallas import tpu as pltpu
from jax.experimental.pallas import tpu_sc as plsc

VOCAB = 1000000
DIM = 64
BATCH = 16384

_NUM_WORKERS = 32
_ROWS_PER_WORKER = BATCH // _NUM_WORKERS
_IDX_CHUNK = 128
_NUM_CHUNKS = _ROWS_PER_WORKER // _IDX_CHUNK


@functools.partial(
    pl.kernel,
    mesh=plsc.VectorSubcoreMesh(core_axis_name="c", subcore_axis_name="s"),
    out_type=jax.ShapeDtypeStruct((BATCH, DIM), jnp.float32),
    scratch_types=[
        pltpu.VMEM((_NUM_CHUNKS, _IDX_CHUNK), jnp.int32),
        pltpu.VMEM((_ROWS_PER_WORKER, DIM), jnp.float32),
        pltpu.SemaphoreType.DMA,
    ],
    compiler_params=pltpu.CompilerParams(use_tc_tiling_on_sc=False),
)
def _gather_kernel(idx_hbm, table_hbm, out_hbm, idx_v, rows_v, sem):
    wid = lax.axis_index("s") * 2 + lax.axis_index("c")
    base = wid * _ROWS_PER_WORKER
    pltpu.sync_copy(idx_hbm.at[pl.ds(wid * _NUM_CHUNKS, _NUM_CHUNKS)], idx_v)
    copies = []
    for j in range(_NUM_CHUNKS):
        copies.append(
            pltpu.async_copy(
                table_hbm.at[idx_v.at[j]],
                rows_v.at[pl.ds(j * _IDX_CHUNK, _IDX_CHUNK)],
                sem,
            )
        )
    for c in copies:
        c.wait()
    pltpu.sync_copy(rows_v, out_hbm.at[pl.ds(base, _ROWS_PER_WORKER)])


def kernel(indices, table):
    idx2d = indices.reshape(_NUM_WORKERS * _NUM_CHUNKS, _IDX_CHUNK)
    return _gather_kernel(idx2d, table)


# speedup vs baseline: 2.5624x; 2.5624x over previous
import functools

import jax
import jax.numpy as jnp
from jax import lax
from jax.experimental import pallas as pl
from jax.experimental.pallas import tpu as pltpu
from jax.experimental.pallas import tpu_sc as plsc

VOCAB = 1000000
DIM = 64
BATCH = 16384

_NUM_WORKERS = 32
_RPW = BATCH // _NUM_WORKERS


@functools.partial(
    pl.kernel,
    mesh=plsc.VectorSubcoreMesh(core_axis_name="c", subcore_axis_name="s"),
    out_type=jax.ShapeDtypeStruct((BATCH, DIM), jnp.float32),
    scratch_types=[
        pltpu.VMEM((_RPW,), jnp.int32),
        pltpu.VMEM((_RPW, DIM), jnp.float32),
        pltpu.SemaphoreType.DMA,
    ],
)
def _gather_kernel(idx_hbm, table_hbm, out_hbm, idx_v, rows_v, sem):
    wid = lax.axis_index("s") * 2 + lax.axis_index("c")
    base = wid * _RPW
    pltpu.sync_copy(idx_hbm.at[pl.ds(base, _RPW)], idx_v)

    def fire(grp, _):
        vec = idx_v[pl.ds(grp * 16, 16)]
        gs = lax.shift_right_logical(vec, 3)
        rs = lax.bitwise_and(vec, 7)
        for l in range(16):
            pltpu.make_async_copy(
                table_hbm.at[gs[l], rs[l]], rows_v.at[grp * 16 + l], sem
            ).start()
        return 0

    lax.fori_loop(0, _RPW // 16, fire, 0)
    pltpu.make_async_copy(out_hbm.at[pl.ds(base, _RPW)], rows_v, sem).wait()
    pltpu.sync_copy(rows_v, out_hbm.at[pl.ds(base, _RPW)])


def kernel(indices, table):
    table3 = table.reshape(VOCAB // 8, 8, DIM)
    return _gather_kernel(indices, table3)
